# Initial kernel scaffold; baseline (speedup 1.0000x reference)
import functools

import jax
import jax.numpy as jnp
from jax import lax
from jax.experimental import pallas as pl
from jax.experimental.pallas import tpu as pltpu
from jax.experimental.pallas import tpu_sc as plsc

N = 10000
E = 320000
F = 128

NC = 2
NS = 16
NW = NC * NS
EPW = E // NW
CH = 80
NCH = EPW // CH
RPT = N // NS
ZR = 125

_MESH = plsc.VectorSubcoreMesh(core_axis_name="c", subcore_axis_name="s")


def _spmm_body(x_hbm, row_hbm, col_hbm, val_hbm, out_hbm,
               colv, rowv, valv, rbuf, zbuf, acc, sem):
    cid = lax.axis_index("c")
    sid = lax.axis_index("s")
    wid = sid * NC + cid

    def zero_body(i, carry):
        zbuf[i // 8, pl.ds((i % 8) * 16, 16)] = jnp.zeros((16,), jnp.float32)
        return carry
    lax.fori_loop(0, ZR * 8, zero_body, 0)
    for j in range(RPT // ZR):
        pltpu.sync_copy(zbuf, acc.at[pl.ds(sid * RPT + j * ZR, ZR)])
    plsc.subcore_barrier()

    ebase = wid * EPW

    def chunk_body(k, carry):
        base = ebase + k * CH
        pltpu.sync_copy(col_hbm.at[pl.ds(base, CH)], colv)
        pltpu.sync_copy(row_hbm.at[pl.ds(base, CH)], rowv)
        pltpu.sync_copy(val_hbm.at[pl.ds(base, CH)], valv)
        pltpu.async_copy(x_hbm.at[colv], rbuf, sem).wait()

        def edge_body(e, c2):
            vv = plsc.load_gather(valv, [jnp.full((16,), e, jnp.int32)])
            for g in range(8):
                rbuf[e, pl.ds(g * 16, 16)] = rbuf[e, pl.ds(g * 16, 16)] * vv
            return c2
        lax.fori_loop(0, CH, edge_body, 0)

        pltpu.sync_copy(rbuf, acc.at[rowv], add=True)
        return carry
    lax.fori_loop(0, NCH, chunk_body, 0)

    plsc.subcore_barrier()
    for j in range(RPT // ZR):
        r0 = sid * RPT + j * ZR
        pltpu.sync_copy(acc.at[pl.ds(r0, ZR)], out_hbm.at[cid, pl.ds(r0, ZR)])


_spmm = pl.kernel(
    _spmm_body,
    out_type=jax.ShapeDtypeStruct((NC, N, F), jnp.float32),
    mesh=_MESH,
    scratch_types=[
        pltpu.VMEM((CH,), jnp.int32),
        pltpu.VMEM((CH,), jnp.int32),
        pltpu.VMEM((CH,), jnp.float32),
        pltpu.VMEM((CH, F), jnp.float32),
        pltpu.VMEM((ZR, F), jnp.float32),
        pltpu.VMEM_SHARED((N, F), jnp.float32),
        pltpu.SemaphoreType.DMA,
    ],
)

_RB = 1000


def _combine_body(p_ref, q_ref, o_ref):
    o_ref[...] = p_ref[0] + q_ref[0]


_combine = pl.pallas_call(
    _combine_body,
    grid=(N // _RB,),
    in_specs=[
        pl.BlockSpec((1, _RB, F), lambda i: (0, i, 0)),
        pl.BlockSpec((1, _RB, F), lambda i: (1, i, 0)),
    ],
    out_specs=pl.BlockSpec((_RB, F), lambda i: (i, 0)),
    out_shape=jax.ShapeDtypeStruct((N, F), jnp.float32),
)


def _final_body(x_ref, t1_ref, q0_ref, q1_ref, w_ref, b_ref, o_ref):
    x = x_ref[...]
    t2 = 2.0 * (q0_ref[0] + q1_ref[0]) - x
    acc = jnp.dot(x, w_ref[0:F, :], preferred_element_type=jnp.float32)
    acc += jnp.dot(t1_ref[...], w_ref[F:2 * F, :],
                   preferred_element_type=jnp.float32)
    acc += jnp.dot(t2, w_ref[2 * F:3 * F, :],
                   preferred_element_type=jnp.float32)
    o_ref[...] = acc + b_ref[...]


_final = pl.pallas_call(
    _final_body,
    grid=(N // _RB,),
    in_specs=[
        pl.BlockSpec((_RB, F), lambda i: (i, 0)),
        pl.BlockSpec((_RB, F), lambda i: (i, 0)),
        pl.BlockSpec((1, _RB, F), lambda i: (0, i, 0)),
        pl.BlockSpec((1, _RB, F), lambda i: (1, i, 0)),
        pl.BlockSpec((3 * F, F), lambda i: (0, 0)),
        pl.BlockSpec((1, F), lambda i: (0, 0)),
    ],
    out_specs=pl.BlockSpec((_RB, F), lambda i: (i, 0)),
    out_shape=jax.ShapeDtypeStruct((N, F), jnp.float32),
)


def kernel(X, edge_index, edge_values, W, b):
    row = edge_index[0]
    col = edge_index[1]
    P = _spmm(X, row, col, edge_values)
    T1 = _combine(P)
    Q = _spmm(T1, row, col, edge_values)
    return _final(X, T1, Q, W, b.reshape(1, F))


# speedup vs baseline: 3.9371x; 3.9371x over previous
import functools

import jax
import jax.numpy as jnp
from jax import lax
from jax.experimental import pallas as pl
from jax.experimental.pallas import tpu as pltpu
from jax.experimental.pallas import tpu_sc as plsc

N = 10000
E = 320000
F = 128

NC = 2
NS = 16
NW = NC * NS
EPW = E // NW
CH = 80
NCH = EPW // CH
NP = 10240
RPT = NP // NS
ZR = 128

_MESH = plsc.VectorSubcoreMesh(core_axis_name="c", subcore_axis_name="s")


def _spmm_body(x_hbm, row_hbm, col_hbm, val_hbm, out_hbm,
               colv, rowv, valv, rbuf, zbuf, acc, sem):
    cid = lax.axis_index("c")
    sid = lax.axis_index("s")
    wid = sid * NC + cid

    def zero_body(i, carry):
        zbuf[i // 8, pl.ds((i % 8) * 16, 16)] = jnp.zeros((16,), jnp.float32)
        return carry
    lax.fori_loop(0, ZR * 8, zero_body, 0)
    for j in range(RPT // ZR):
        pltpu.sync_copy(zbuf, acc.at[pl.ds(sid * RPT + j * ZR, ZR)])
    plsc.subcore_barrier()

    ebase = wid * EPW

    def chunk_body(k, carry):
        base = ebase + k * CH
        pltpu.sync_copy(col_hbm.at[pl.ds(base, CH)], colv)
        pltpu.sync_copy(row_hbm.at[pl.ds(base, CH)], rowv)
        pltpu.sync_copy(val_hbm.at[pl.ds(base, CH)], valv)
        pltpu.async_copy(x_hbm.at[colv], rbuf, sem).wait()

        def grp_body(i, c2):
            vv16 = valv[pl.ds(i * 16, 16)]
            for j in range(16):
                e = i * 16 + j
                vv = jnp.full((16,), vv16[j], jnp.float32)
                for g in range(8):
                    rbuf[e, pl.ds(g * 16, 16)] = rbuf[e, pl.ds(g * 16, 16)] * vv
            return c2
        lax.fori_loop(0, CH // 16, grp_body, 0)

        pltpu.sync_copy(rbuf, acc.at[rowv], add=True)
        return carry
    lax.fori_loop(0, NCH, chunk_body, 0)

    plsc.subcore_barrier()
    for j in range(RPT // ZR):
        r0 = sid * RPT + j * ZR
        pltpu.sync_copy(acc.at[pl.ds(r0, ZR)], out_hbm.at[cid, pl.ds(r0, ZR)])


_spmm = pl.kernel(
    _spmm_body,
    out_type=jax.ShapeDtypeStruct((NC, NP, F), jnp.float32),
    mesh=_MESH,
    scratch_types=[
        pltpu.VMEM((CH,), jnp.int32),
        pltpu.VMEM((CH,), jnp.int32),
        pltpu.VMEM((CH,), jnp.float32),
        pltpu.VMEM((CH, F), jnp.float32),
        pltpu.VMEM((ZR, F), jnp.float32),
        pltpu.VMEM_SHARED((NP, F), jnp.float32),
        pltpu.SemaphoreType.DMA,
    ],
)

_RB = 1000


def _combine_body(p_ref, o_ref):
    o_ref[...] = p_ref[0] + p_ref[1]


_combine = pl.pallas_call(
    _combine_body,
    grid=(N // _RB,),
    in_specs=[
        pl.BlockSpec((2, _RB, F), lambda i: (0, i, 0)),
    ],
    out_specs=pl.BlockSpec((_RB, F), lambda i: (i, 0)),
    out_shape=jax.ShapeDtypeStruct((N, F), jnp.float32),
)


def _final_body(x_ref, t1_ref, q_ref, w_ref, b_ref, o_ref):
    x = x_ref[...]
    t2 = 2.0 * (q_ref[0] + q_ref[1]) - x
    acc = jnp.dot(x, w_ref[0:F, :], preferred_element_type=jnp.float32)
    acc += jnp.dot(t1_ref[...], w_ref[F:2 * F, :],
                   preferred_element_type=jnp.float32)
    acc += jnp.dot(t2, w_ref[2 * F:3 * F, :],
                   preferred_element_type=jnp.float32)
    o_ref[...] = acc + b_ref[...]


_final = pl.pallas_call(
    _final_body,
    grid=(N // _RB,),
    in_specs=[
        pl.BlockSpec((_RB, F), lambda i: (i, 0)),
        pl.BlockSpec((_RB, F), lambda i: (i, 0)),
        pl.BlockSpec((2, _RB, F), lambda i: (0, i, 0)),
        pl.BlockSpec((3 * F, F), lambda i: (0, 0)),
        pl.BlockSpec((1, F), lambda i: (0, 0)),
    ],
    out_specs=pl.BlockSpec((_RB, F), lambda i: (i, 0)),
    out_shape=jax.ShapeDtypeStruct((N, F), jnp.float32),
)


def kernel(X, edge_index, edge_values, W, b):
    row = edge_index[0]
    col = edge_index[1]
    P = _spmm(X, row, col, edge_values)
    T1 = _combine(P)
    Q = _spmm(T1, row, col, edge_values)
    return _final(X, T1, Q, W, b.reshape(1, F))


# speedup vs baseline: 10.7639x; 2.7339x over previous
import functools

import jax
import jax.numpy as jnp
from jax import lax
from jax.experimental import pallas as pl
from jax.experimental.pallas import tpu as pltpu
from jax.experimental.pallas import tpu_sc as plsc

N = 10000
E = 320000
F = 128

NC = 2
NS = 16
NW = NC * NS
EPW = E // NW
CH = 80
NCH = EPW // CH
NB = 4
NP = 10240
RPT = NP // NS

_MESH = plsc.VectorSubcoreMesh(core_axis_name="c", subcore_axis_name="s")


def _spmm_body(x_hbm, edata_hbm, vdata_hbm, out_hbm,
               ebuf, vbuf, rbuf, acc,
               es0, es1, es2, es3, vs0, vs1, vs2, vs3,
               gs0, gs1, gs2, gs3, ss0, ss1, ss2, ss3):
    esem = (es0, es1, es2, es3)
    vsem = (vs0, vs1, vs2, vs3)
    gsem = (gs0, gs1, gs2, gs3)
    ssem = (ss0, ss1, ss2, ss3)
    cid = lax.axis_index("c")
    sid = lax.axis_index("s")
    wid = sid * NC + cid

    def zero_body(i, carry):
        rbuf[0, i // 8, pl.ds((i % 8) * 16, 16)] = jnp.zeros((16,), jnp.float32)
        return carry
    lax.fori_loop(0, CH * 8, zero_body, 0)
    for j in range(RPT // CH):
        pltpu.sync_copy(rbuf.at[0], acc.at[pl.ds(sid * RPT + j * CH, CH)])
    plsc.subcore_barrier()

    def idx_start(c, b):
        pltpu.async_copy(edata_hbm.at[wid, c], ebuf.at[b], esem[b])
        pltpu.async_copy(vdata_hbm.at[wid, c], vbuf.at[b], vsem[b])

    def idx_wait(c, b):
        pltpu.make_async_copy(edata_hbm.at[wid, c], ebuf.at[b],
                              esem[b]).wait()

    def val_wait(c, b):
        pltpu.make_async_copy(vdata_hbm.at[wid, c], vbuf.at[b],
                              vsem[b]).wait()

    def gather_start(c, b):
        pltpu.async_copy(x_hbm.at[ebuf.at[b, 0]], rbuf.at[b], gsem[b])

    def gather_wait(c, b):
        pltpu.make_async_copy(x_hbm.at[ebuf.at[b, 0]], rbuf.at[b],
                              gsem[b]).wait()

    def scatter_start(c, b):
        pltpu.async_copy(rbuf.at[b], acc.at[ebuf.at[b, 1]], ssem[b], add=True)

    def scatter_wait(c, b):
        pltpu.make_async_copy(rbuf.at[b], acc.at[ebuf.at[b, 1]],
                              ssem[b]).wait()

    def process(c, b):
        gather_wait(c, b)
        val_wait(c, b)

        def grp_body(i, c2):
            vv16 = vbuf[b, 0, pl.ds(i * 16, 16)]
            for j in range(16):
                e = i * 16 + j
                vv = jnp.full((16,), vv16[j], jnp.float32)
                for g in range(8):
                    rbuf[b, e, pl.ds(g * 16, 16)] = (
                        rbuf[b, e, pl.ds(g * 16, 16)] * vv)
            return c2
        lax.fori_loop(0, CH // 16, grp_body, 0)
        scatter_start(c, b)

    for c in range(NB):
        idx_start(c, c)
    for c in range(2):
        idx_wait(c, c)
        gather_start(c, c)

    def ring_body(k, carry):
        for b in range(NB):
            c = k * NB + b
            process(c, b)

            @pl.when(c + 2 < NCH)
            def _pre_gather():
                idx_wait(c + 2, (b + 2) % NB)
                gather_start(c + 2, (b + 2) % NB)

            @pl.when(jnp.logical_and(c >= 1, c + 3 < NCH))
            def _pre_idx():
                scatter_wait(c - 1, (b - 1) % NB)
                idx_start(c + 3, (b + 3) % NB)
        return carry
    lax.fori_loop(0, NCH // NB, ring_body, 0)

    process(NCH - 1, (NCH - 1) % NB)
    for c in range(NCH - NB, NCH):
        scatter_wait(c, c % NB)

    plsc.subcore_barrier()
    for j in range(RPT // 128):
        r0 = sid * RPT + j * 128
        pltpu.sync_copy(acc.at[pl.ds(r0, 128)], out_hbm.at[cid, pl.ds(r0, 128)])


_spmm = pl.kernel(
    _spmm_body,
    out_type=jax.ShapeDtypeStruct((NC, NP, F), jnp.float32),
    mesh=_MESH,
    scratch_types=[
        pltpu.VMEM((NB, 2, CH), jnp.int32),
        pltpu.VMEM((NB, 1, CH), jnp.float32),
        pltpu.VMEM((NB, CH, F), jnp.float32),
        pltpu.VMEM_SHARED((NP, F), jnp.float32),
    ] + [pltpu.SemaphoreType.DMA] * (4 * NB),
)

_RB = 1000


def _combine_body(p_ref, o_ref):
    o_ref[...] = p_ref[0] + p_ref[1]


_combine = pl.pallas_call(
    _combine_body,
    grid=(N // _RB,),
    in_specs=[
        pl.BlockSpec((2, _RB, F), lambda i: (0, i, 0)),
    ],
    out_specs=pl.BlockSpec((_RB, F), lambda i: (i, 0)),
    out_shape=jax.ShapeDtypeStruct((N, F), jnp.float32),
)


def _final_body(x_ref, t1_ref, q_ref, w_ref, b_ref, o_ref):
    x = x_ref[...]
    t2 = 2.0 * (q_ref[0] + q_ref[1]) - x
    acc = jnp.dot(x, w_ref[0:F, :], preferred_element_type=jnp.float32)
    acc += jnp.dot(t1_ref[...], w_ref[F:2 * F, :],
                   preferred_element_type=jnp.float32)
    acc += jnp.dot(t2, w_ref[2 * F:3 * F, :],
                   preferred_element_type=jnp.float32)
    o_ref[...] = acc + b_ref[...]


_final = pl.pallas_call(
    _final_body,
    grid=(N // _RB,),
    in_specs=[
        pl.BlockSpec((_RB, F), lambda i: (i, 0)),
        pl.BlockSpec((_RB, F), lambda i: (i, 0)),
        pl.BlockSpec((2, _RB, F), lambda i: (0, i, 0)),
        pl.BlockSpec((3 * F, F), lambda i: (0, 0)),
        pl.BlockSpec((1, F), lambda i: (0, 0)),
    ],
    out_specs=pl.BlockSpec((_RB, F), lambda i: (i, 0)),
    out_shape=jax.ShapeDtypeStruct((N, F), jnp.float32),
)


def kernel(X, edge_index, edge_values, W, b):
    col = edge_index[1].reshape(NW, NCH, CH)
    row = edge_index[0].reshape(NW, NCH, CH)
    edata = jnp.stack([col, row], axis=2)
    vdata = edge_values.reshape(NW, NCH, 1, CH)
    P = _spmm(X, edata, vdata)
    T1 = _combine(P)
    Q = _spmm(T1, edata, vdata)
    return _final(X, T1, Q, W, b.reshape(1, F))


# speedup vs baseline: 10.8032x; 1.0037x over previous
import jax
import jax.numpy as jnp
from jax import lax
from jax.experimental import pallas as pl
from jax.experimental.pallas import tpu as pltpu
from jax.experimental.pallas import tpu_sc as plsc

N = 10000
E = 320000
F = 128

NC = 2
NS = 16
NW = NC * NS
EPW = E // NW
CH = 80
NCH = EPW // CH
NB = 4
NP = 10240
RPT = NP // NS

_MESH = plsc.VectorSubcoreMesh(core_axis_name="c", subcore_axis_name="s")


def _spmm_body(x_hbm, edata_hbm, vdata_hbm, out_hbm,
               ebuf, vbuf, rbuf, acc,
               es0, es1, es2, es3, vs0, vs1, vs2, vs3,
               gs0, gs1, gs2, gs3, ss0, ss1, ss2, ss3):
    esem = (es0, es1, es2, es3)
    vsem = (vs0, vs1, vs2, vs3)
    gsem = (gs0, gs1, gs2, gs3)
    ssem = (ss0, ss1, ss2, ss3)
    cid = lax.axis_index("c")
    sid = lax.axis_index("s")
    wid = sid * NC + cid

    def zero_body(i, carry):
        rbuf[0, i // 8, pl.ds((i % 8) * 16, 16)] = jnp.zeros((16,), jnp.float32)
        return carry
    lax.fori_loop(0, CH * 8, zero_body, 0)
    for j in range(RPT // CH):
        pltpu.sync_copy(rbuf.at[0], acc.at[pl.ds(sid * RPT + j * CH, CH)])
    plsc.subcore_barrier()

    def idx_start(c, b):
        pltpu.async_copy(edata_hbm.at[wid, c], ebuf.at[b], esem[b])
        pltpu.async_copy(vdata_hbm.at[wid, c], vbuf.at[b], vsem[b])

    def idx_wait(c, b):
        pltpu.make_async_copy(edata_hbm.at[wid, c], ebuf.at[b],
                              esem[b]).wait()

    def val_wait(c, b):
        pltpu.make_async_copy(vdata_hbm.at[wid, c], vbuf.at[b],
                              vsem[b]).wait()

    def gather_start(c, b):
        pltpu.async_copy(x_hbm.at[ebuf.at[b, 0]], rbuf.at[b], gsem[b])

    def gather_wait(c, b):
        pltpu.make_async_copy(x_hbm.at[ebuf.at[b, 0]], rbuf.at[b],
                              gsem[b]).wait()

    def scatter_start(c, b):
        pltpu.async_copy(rbuf.at[b], acc.at[ebuf.at[b, 1]], ssem[b], add=True)

    def scatter_wait(c, b):
        pltpu.make_async_copy(rbuf.at[b], acc.at[ebuf.at[b, 1]],
                              ssem[b]).wait()

    def process(c, b):
        gather_wait(c, b)
        val_wait(c, b)

        def grp_body(i, c2):
            vv16 = vbuf[b, 0, pl.ds(i * 16, 16)]
            for j in range(16):
                e = i * 16 + j
                vv = jnp.full((16,), vv16[j], jnp.float32)
                for g in range(8):
                    rbuf[b, e, pl.ds(g * 16, 16)] = (
                        rbuf[b, e, pl.ds(g * 16, 16)] * vv)
            return c2
        lax.fori_loop(0, CH // 16, grp_body, 0)
        scatter_start(c, b)

    for c in range(NB):
        idx_start(c, c)
    for c in range(2):
        idx_wait(c, c)
        gather_start(c, c)

    def ring_body(k, carry):
        for b in range(NB):
            c = k * NB + b
            process(c, b)

            @pl.when(c + 2 < NCH)
            def _pre_gather():
                idx_wait(c + 2, (b + 2) % NB)
                gather_start(c + 2, (b + 2) % NB)

            @pl.when(jnp.logical_and(c >= 1, c + 3 < NCH))
            def _pre_idx():
                scatter_wait(c - 1, (b - 1) % NB)
                idx_start(c + 3, (b + 3) % NB)
        return carry
    lax.fori_loop(0, NCH // NB, ring_body, 0)

    process(NCH - 1, (NCH - 1) % NB)
    for c in range(NCH - NB, NCH):
        scatter_wait(c, c % NB)

    plsc.subcore_barrier()
    for j in range(RPT // 128):
        r0 = sid * RPT + j * 128
        pltpu.sync_copy(acc.at[pl.ds(r0, 128)], out_hbm.at[cid, pl.ds(r0, 128)])


_spmm = pl.kernel(
    _spmm_body,
    out_type=jax.ShapeDtypeStruct((NC, NP, F), jnp.float32),
    mesh=_MESH,
    scratch_types=[
        pltpu.VMEM((NB, 2, CH), jnp.int32),
        pltpu.VMEM((NB, 1, CH), jnp.float32),
        pltpu.VMEM((NB, CH, F), jnp.float32),
        pltpu.VMEM_SHARED((NP, F), jnp.float32),
    ] + [pltpu.SemaphoreType.DMA] * (4 * NB),
)

_RB = 1000


def _combine_body(p_ref, o_ref):
    o_ref[...] = p_ref[0] + p_ref[1]


_combine = pl.pallas_call(
    _combine_body,
    grid=(N // _RB,),
    in_specs=[
        pl.BlockSpec((2, _RB, F), lambda i: (0, i, 0)),
    ],
    out_specs=pl.BlockSpec((_RB, F), lambda i: (i, 0)),
    out_shape=jax.ShapeDtypeStruct((N, F), jnp.float32),
)


def _final_body(x_ref, t1_ref, q_ref, w_ref, b_ref, o_ref):
    x = x_ref[...]
    t2 = 2.0 * (q_ref[0] + q_ref[1]) - x
    acc = jnp.dot(x, w_ref[0:F, :], preferred_element_type=jnp.float32)
    acc += jnp.dot(t1_ref[...], w_ref[F:2 * F, :],
                   preferred_element_type=jnp.float32)
    acc += jnp.dot(t2, w_ref[2 * F:3 * F, :],
                   preferred_element_type=jnp.float32)
    o_ref[...] = acc + b_ref[...]


_final = pl.pallas_call(
    _final_body,
    grid=(N // _RB,),
    in_specs=[
        pl.BlockSpec((_RB, F), lambda i: (i, 0)),
        pl.BlockSpec((_RB, F), lambda i: (i, 0)),
        pl.BlockSpec((2, _RB, F), lambda i: (0, i, 0)),
        pl.BlockSpec((3 * F, F), lambda i: (0, 0)),
        pl.BlockSpec((1, F), lambda i: (0, 0)),
    ],
    out_specs=pl.BlockSpec((_RB, F), lambda i: (i, 0)),
    out_shape=jax.ShapeDtypeStruct((N, F), jnp.float32),
)


def kernel(X, edge_index, edge_values, W, b):
    col = edge_index[1].reshape(NW, NCH, CH)
    row = edge_index[0].reshape(NW, NCH, CH)
    edata = jnp.stack([col, row], axis=2)
    vdata = edge_values.reshape(NW, NCH, 1, CH)
    P = _spmm(X, edata, vdata)
    T1 = _combine(P)
    Q = _spmm(T1, edata, vdata)
    return _final(X, T1, Q, W, b.reshape(1, F))
